# baseline (speedup 1.0000x reference)
import jax
import jax.numpy as jnp
from jax.experimental import pallas as pl
from jax.experimental.pallas import tpu as pltpu

_B, _H, _S, _D = 8, 16, 2048, 128
_L = 16
_R = _B * _H * _S
_RV = _B * _H * _L
_BLKR = 8192
_NBLK = _R // _BLKR
_NT = 2 * _NBLK
_SLABS_PB = _BLKR // _S
_VROWS_PB = _SLABS_PB * _L
_NBUF = 4


def _body(pos_ref, kc, vc, kv, vv, ko, vo,
          buf0, buf1, buf2, buf3, kvbuf, vvbuf,
          rs0, rs1, rs2, rs3, ws0, ws1, ws2, ws3, vs):
    p0 = pos_ref[0]
    kvread = pltpu.make_async_copy(kv, kvbuf, vs)
    vvread = pltpu.make_async_copy(vv, vvbuf, vs)
    kvread.start()
    vvread.start()
    kvread.wait()
    vvread.wait()

    bufs = (buf0, buf1, buf2, buf3)
    rsems = (rs0, rs1, rs2, rs3)
    wsems = (ws0, ws1, ws2, ws3)
    reads = [None] * _NBUF
    writes = [None] * _NBUF

    def src_dst_vals(t):
        if t < _NBLK:
            return kc, ko, kvbuf, t
        return vc, vo, vvbuf, t - _NBLK

    def start_read(t):
        src, _, _, tt = src_dst_vals(t)
        slot = t % _NBUF
        reads[slot] = pltpu.make_async_copy(
            src.at[pl.ds(tt * _BLKR, _BLKR)], bufs[slot], rsems[slot])
        reads[slot].start()

    for t in range(_NBUF - 1):
        start_read(t)
    for t in range(_NT):
        slot = t % _NBUF
        pre = t + _NBUF - 1
        if pre < _NT:
            pslot = pre % _NBUF
            if writes[pslot] is not None:
                writes[pslot].wait()
                writes[pslot] = None
            start_read(pre)
        _, dst, valbuf, tt = src_dst_vals(t)
        reads[slot].wait()
        for s in range(_SLABS_PB):
            bufs[slot][pl.ds(s * _S + p0, _L), :] = (
                valbuf[pl.ds(tt * _VROWS_PB + s * _L, _L), :])
        writes[slot] = pltpu.make_async_copy(
            bufs[slot], dst.at[pl.ds(tt * _BLKR, _BLKR)], wsems[slot])
        writes[slot].start()
    for j in range(_NBUF):
        if writes[j] is not None:
            writes[j].wait()


def kernel(k_cache, v_cache, input_pos, k_val, v_val):
    hbm = pl.BlockSpec(memory_space=pltpu.HBM)
    out = pl.pallas_call(
        _body,
        grid_spec=pltpu.PrefetchScalarGridSpec(
            num_scalar_prefetch=1,
            grid=(1,),
            in_specs=[hbm, hbm, hbm, hbm],
            out_specs=[hbm, hbm],
            scratch_shapes=(
                [pltpu.VMEM((_BLKR, _D), jnp.float32)] * _NBUF
                + [pltpu.VMEM((_RV, _D), jnp.float32)] * 2
                + [pltpu.SemaphoreType.DMA] * (2 * _NBUF + 1)
            ),
        ),
        out_shape=[jax.ShapeDtypeStruct((_R, _D), jnp.float32)] * 2,
    )(input_pos,
      k_cache.reshape(_R, _D), v_cache.reshape(_R, _D),
      k_val.reshape(_RV, _D), v_val.reshape(_RV, _D))
    return (out[0].reshape(_B, _H, _S, _D), out[1].reshape(_B, _H, _S, _D))
